# Initial kernel scaffold; baseline (speedup 1.0000x reference)
import jax
import jax.numpy as jnp
from jax.experimental import pallas as pl


def kernel(x, edge_index, W1, b1, W2, b2):
    raise NotImplementedError("write your pallas kernel here")


# speedup vs baseline: 9.6132x; 9.6132x over previous
import functools

import jax
import jax.numpy as jnp
from jax import lax
from jax.experimental import pallas as pl
from jax.experimental.pallas import tpu as pltpu
from jax.experimental.pallas import tpu_sc as plsc

N = 10000
E = 320000
D = 128

NC = 2
NS = 16
NW = NC * NS
NPAD = 10240
RPT = NPAD // NS
CH = 128
EPT = 10240
NCHUNK = EPT // CH
EPAD = NW * EPT

_mesh = plsc.VectorSubcoreMesh(core_axis_name="c", subcore_axis_name="s")


@functools.partial(
    pl.kernel,
    out_type=jax.ShapeDtypeStruct((NC, NPAD), jnp.float32),
    mesh=_mesh,
    scratch_types=[
        pltpu.VMEM((NCHUNK, CH), jnp.int32),
        pltpu.VMEM((CH,), jnp.float32),
        pltpu.VMEM_SHARED((NPAD,), jnp.float32),
    ],
)
def _deg_kernel(dst_hbm, ones_hbm, zeros_hbm, deg_out, didx_v, ones_v, acc_sh):
    c = lax.axis_index("c")
    s = lax.axis_index("s")
    wid = c * NS + s
    pltpu.sync_copy(zeros_hbm, acc_sh.at[pl.ds(s * RPT, RPT)])
    pltpu.sync_copy(ones_hbm, ones_v)
    pltpu.sync_copy(dst_hbm.at[wid], didx_v)
    plsc.subcore_barrier()

    @pl.loop(0, NCHUNK)
    def _(i):
        pltpu.sync_copy(ones_v, acc_sh.at[didx_v.at[i]], add=True)

    plsc.subcore_barrier()
    pltpu.sync_copy(acc_sh.at[pl.ds(s * RPT, RPT)],
                    deg_out.at[c, pl.ds(s * RPT, RPT)])


@functools.partial(
    pl.kernel,
    out_type=jax.ShapeDtypeStruct((NC, NPAD, D), jnp.float32),
    mesh=_mesh,
    scratch_types=[
        pltpu.VMEM((NCHUNK, CH), jnp.int32),
        pltpu.VMEM((NCHUNK, CH), jnp.int32),
        pltpu.VMEM((CH, D), jnp.float32),
        pltpu.VMEM_SHARED((NPAD, D), jnp.float32),
    ],
)
def _agg_kernel(g_hbm, src_hbm, dst_hbm, zrows_hbm, parts_out,
                sidx_v, didx_v, rows_v, acc_sh):
    c = lax.axis_index("c")
    s = lax.axis_index("s")
    wid = c * NS + s

    @pl.loop(0, RPT // CH)
    def _(j):
        pltpu.sync_copy(zrows_hbm, acc_sh.at[pl.ds(s * RPT + j * CH, CH)])

    pltpu.sync_copy(src_hbm.at[wid], sidx_v)
    pltpu.sync_copy(dst_hbm.at[wid], didx_v)
    plsc.subcore_barrier()

    @pl.loop(0, NCHUNK)
    def _(i):
        pltpu.sync_copy(g_hbm.at[sidx_v.at[i]], rows_v)
        pltpu.sync_copy(rows_v, acc_sh.at[didx_v.at[i]], add=True)

    plsc.subcore_barrier()

    @pl.loop(0, RPT // CH)
    def _(j):
        r = s * RPT + j * CH
        pltpu.sync_copy(acc_sh.at[pl.ds(r, CH)], parts_out.at[c, pl.ds(r, CH)])


BLK = 1024


def _k1_body(parts_ref, x_ref, w_ref, g_ref, dinv_ref):
    deg = parts_ref[0, :] + parts_ref[1, :] + 1.0
    dinv = lax.rsqrt(deg)
    h = jnp.dot(x_ref[...], w_ref[...], preferred_element_type=jnp.float32)
    g_ref[...] = h * dinv[:, None]
    dinv_ref[...] = dinv[:, None]


def _k1(parts, x_p, w1):
    return pl.pallas_call(
        _k1_body,
        grid=(NPAD // BLK,),
        in_specs=[
            pl.BlockSpec((NC, BLK), lambda i: (0, i)),
            pl.BlockSpec((BLK, D), lambda i: (i, 0)),
            pl.BlockSpec((D, D), lambda i: (0, 0)),
        ],
        out_specs=[
            pl.BlockSpec((BLK, D), lambda i: (i, 0)),
            pl.BlockSpec((BLK, 1), lambda i: (i, 0)),
        ],
        out_shape=[
            jax.ShapeDtypeStruct((NPAD, D), jnp.float32),
            jax.ShapeDtypeStruct((NPAD, 1), jnp.float32),
        ],
    )(parts, x_p, w1)


def _kmid_body(parts_ref, g1_ref, dinv_ref, b1_ref, w2_ref, g2_ref):
    a = parts_ref[0] + parts_ref[1] + g1_ref[...]
    dinv = dinv_ref[...]
    z = jnp.maximum(a * dinv + b1_ref[...], 0.0)
    g2_ref[...] = jnp.dot(z, w2_ref[...],
                          preferred_element_type=jnp.float32) * dinv


def _kmid(parts1, g1, dinv, b1, w2):
    return pl.pallas_call(
        _kmid_body,
        grid=(NPAD // BLK,),
        in_specs=[
            pl.BlockSpec((NC, BLK, D), lambda i: (0, i, 0)),
            pl.BlockSpec((BLK, D), lambda i: (i, 0)),
            pl.BlockSpec((BLK, 1), lambda i: (i, 0)),
            pl.BlockSpec((1, D), lambda i: (0, 0)),
            pl.BlockSpec((D, D), lambda i: (0, 0)),
        ],
        out_specs=pl.BlockSpec((BLK, D), lambda i: (i, 0)),
        out_shape=jax.ShapeDtypeStruct((NPAD, D), jnp.float32),
    )(parts1, g1, dinv, b1, w2)


def _kout_body(parts_ref, g2_ref, dinv_ref, b2_ref, out_ref):
    a = parts_ref[0] + parts_ref[1] + g2_ref[...]
    out_ref[...] = a * dinv_ref[...] + b2_ref[...]


def _kout(parts2, g2, dinv, b2):
    return pl.pallas_call(
        _kout_body,
        grid=(NPAD // BLK,),
        in_specs=[
            pl.BlockSpec((NC, BLK, D), lambda i: (0, i, 0)),
            pl.BlockSpec((BLK, D), lambda i: (i, 0)),
            pl.BlockSpec((BLK, 1), lambda i: (i, 0)),
            pl.BlockSpec((1, D), lambda i: (0, 0)),
        ],
        out_specs=pl.BlockSpec((BLK, D), lambda i: (i, 0)),
        out_shape=jax.ShapeDtypeStruct((NPAD, D), jnp.float32),
    )(parts2, g2, dinv, b2)


def kernel(x, edge_index, W1, b1, W2, b2):
    src = edge_index[0]
    dst = edge_index[1]
    pad = jnp.full((EPAD - E,), N, dtype=jnp.int32)
    srcr = jnp.concatenate([src.astype(jnp.int32), pad]).reshape(NW, NCHUNK, CH)
    dstr = jnp.concatenate([dst.astype(jnp.int32), pad]).reshape(NW, NCHUNK, CH)
    x_p = jnp.zeros((NPAD, D), jnp.float32).at[:N].set(x)
    ones_ch = jnp.ones((CH,), jnp.float32)
    zeros_rpt = jnp.zeros((RPT,), jnp.float32)
    zrows = jnp.zeros((CH, D), jnp.float32)

    deg_parts = _deg_kernel(dstr, ones_ch, zeros_rpt)
    g1, dinv = _k1(deg_parts, x_p, W1)
    parts1 = _agg_kernel(g1, srcr, dstr, zrows)
    g2 = _kmid(parts1, g1, dinv, b1.reshape(1, D), W2)
    parts2 = _agg_kernel(g2, srcr, dstr, zrows)
    out = _kout(parts2, g2, dinv, b2.reshape(1, D))
    return out[:N]


# speedup vs baseline: 11.0115x; 1.1455x over previous
import functools

import jax
import jax.numpy as jnp
from jax import lax
from jax.experimental import pallas as pl
from jax.experimental.pallas import tpu as pltpu
from jax.experimental.pallas import tpu_sc as plsc

N = 10000
E = 320000
D = 128

NC = 2
NS = 16
NW = NC * NS
NPAD = 10240
RPT = NPAD // NS
CH = 128
EPT = 10240
NCHUNK = EPT // CH
EPAD = NW * EPT

_mesh = plsc.VectorSubcoreMesh(core_axis_name="c", subcore_axis_name="s")


@functools.partial(
    pl.kernel,
    out_type=jax.ShapeDtypeStruct((NC, NPAD), jnp.float32),
    mesh=_mesh,
    scratch_types=[
        pltpu.VMEM((NCHUNK, CH), jnp.int32),
        pltpu.VMEM((CH,), jnp.float32),
        pltpu.VMEM_SHARED((NPAD,), jnp.float32),
    ],
)
def _deg_kernel(dst_hbm, ones_hbm, zeros_hbm, deg_out, didx_v, ones_v, acc_sh):
    c = lax.axis_index("c")
    s = lax.axis_index("s")
    wid = c * NS + s
    pltpu.sync_copy(zeros_hbm, acc_sh.at[pl.ds(s * RPT, RPT)])
    pltpu.sync_copy(ones_hbm, ones_v)
    pltpu.sync_copy(dst_hbm.at[wid], didx_v)
    plsc.subcore_barrier()

    @pl.loop(0, NCHUNK)
    def _(i):
        pltpu.sync_copy(ones_v, acc_sh.at[didx_v.at[i]], add=True)

    plsc.subcore_barrier()
    pltpu.sync_copy(acc_sh.at[pl.ds(s * RPT, RPT)],
                    deg_out.at[c, pl.ds(s * RPT, RPT)])


@functools.partial(
    pl.kernel,
    out_type=jax.ShapeDtypeStruct((NC, NPAD, D), jnp.float32),
    mesh=_mesh,
    scratch_types=[
        pltpu.VMEM((NCHUNK, CH), jnp.int32),
        pltpu.VMEM((2, CH), jnp.int32),
        pltpu.VMEM((CH, D), jnp.float32),
        pltpu.VMEM((CH, D), jnp.float32),
        pltpu.VMEM_SHARED((NPAD, D), jnp.float32),
        pltpu.SemaphoreType.DMA,
        pltpu.SemaphoreType.DMA,
        pltpu.SemaphoreType.DMA,
        pltpu.SemaphoreType.DMA,
    ],
)
def _agg_kernel(g_hbm, src_hbm, dst_hbm, zrows_hbm, parts_out,
                sidx_v, didx_v, rows0_v, rows1_v, acc_sh,
                sem0, sem1, semd0, semd1):
    c = lax.axis_index("c")
    s = lax.axis_index("s")
    wid = c * NS + s

    @pl.loop(0, RPT // CH)
    def _(j):
        pltpu.sync_copy(zrows_hbm, acc_sh.at[pl.ds(s * RPT + j * CH, CH)])

    pltpu.sync_copy(src_hbm.at[wid], sidx_v)
    plsc.subcore_barrier()

    pltpu.async_copy(g_hbm.at[sidx_v.at[0]], rows0_v, sem0)
    pltpu.async_copy(dst_hbm.at[wid, 0], didx_v.at[0], semd0)
    pltpu.async_copy(g_hbm.at[sidx_v.at[1]], rows1_v, sem1)
    pltpu.async_copy(dst_hbm.at[wid, 1], didx_v.at[1], semd1)

    @pl.loop(0, NCHUNK // 2 - 1)
    def _(p):
        i = p * 2
        pltpu.make_async_copy(g_hbm.at[sidx_v.at[i]], rows0_v, sem0).wait()
        pltpu.make_async_copy(dst_hbm.at[wid, i], didx_v.at[0], semd0).wait()
        pltpu.sync_copy(rows0_v, acc_sh.at[didx_v.at[0]], add=True)
        pltpu.async_copy(g_hbm.at[sidx_v.at[i + 2]], rows0_v, sem0)
        pltpu.async_copy(dst_hbm.at[wid, i + 2], didx_v.at[0], semd0)
        pltpu.make_async_copy(g_hbm.at[sidx_v.at[i + 1]], rows1_v, sem1).wait()
        pltpu.make_async_copy(dst_hbm.at[wid, i + 1], didx_v.at[1], semd1).wait()
        pltpu.sync_copy(rows1_v, acc_sh.at[didx_v.at[1]], add=True)
        pltpu.async_copy(g_hbm.at[sidx_v.at[i + 3]], rows1_v, sem1)
        pltpu.async_copy(dst_hbm.at[wid, i + 3], didx_v.at[1], semd1)

    i_last = NCHUNK - 2
    pltpu.make_async_copy(g_hbm.at[sidx_v.at[i_last]], rows0_v, sem0).wait()
    pltpu.make_async_copy(dst_hbm.at[wid, i_last], didx_v.at[0], semd0).wait()
    pltpu.sync_copy(rows0_v, acc_sh.at[didx_v.at[0]], add=True)
    pltpu.make_async_copy(g_hbm.at[sidx_v.at[i_last + 1]], rows1_v, sem1).wait()
    pltpu.make_async_copy(dst_hbm.at[wid, i_last + 1], didx_v.at[1], semd1).wait()
    pltpu.sync_copy(rows1_v, acc_sh.at[didx_v.at[1]], add=True)

    plsc.subcore_barrier()

    @pl.loop(0, RPT // CH)
    def _(j):
        r = s * RPT + j * CH
        pltpu.sync_copy(acc_sh.at[pl.ds(r, CH)], parts_out.at[c, pl.ds(r, CH)])


BLK = 1024


def _k1_body(parts_ref, x_ref, w_ref, g_ref, dinv_ref):
    deg = parts_ref[0, :] + parts_ref[1, :] + 1.0
    dinv = lax.rsqrt(deg)
    h = jnp.dot(x_ref[...], w_ref[...], preferred_element_type=jnp.float32)
    g_ref[...] = h * dinv[:, None]
    dinv_ref[...] = dinv[:, None]


def _k1(parts, x_p, w1):
    return pl.pallas_call(
        _k1_body,
        grid=(NPAD // BLK,),
        in_specs=[
            pl.BlockSpec((NC, BLK), lambda i: (0, i)),
            pl.BlockSpec((BLK, D), lambda i: (i, 0)),
            pl.BlockSpec((D, D), lambda i: (0, 0)),
        ],
        out_specs=[
            pl.BlockSpec((BLK, D), lambda i: (i, 0)),
            pl.BlockSpec((BLK, 1), lambda i: (i, 0)),
        ],
        out_shape=[
            jax.ShapeDtypeStruct((NPAD, D), jnp.float32),
            jax.ShapeDtypeStruct((NPAD, 1), jnp.float32),
        ],
    )(parts, x_p, w1)


def _kmid_body(parts_ref, g1_ref, dinv_ref, b1_ref, w2_ref, g2_ref):
    a = parts_ref[0] + parts_ref[1] + g1_ref[...]
    dinv = dinv_ref[...]
    z = jnp.maximum(a * dinv + b1_ref[...], 0.0)
    g2_ref[...] = jnp.dot(z, w2_ref[...],
                          preferred_element_type=jnp.float32) * dinv


def _kmid(parts1, g1, dinv, b1, w2):
    return pl.pallas_call(
        _kmid_body,
        grid=(NPAD // BLK,),
        in_specs=[
            pl.BlockSpec((NC, BLK, D), lambda i: (0, i, 0)),
            pl.BlockSpec((BLK, D), lambda i: (i, 0)),
            pl.BlockSpec((BLK, 1), lambda i: (i, 0)),
            pl.BlockSpec((1, D), lambda i: (0, 0)),
            pl.BlockSpec((D, D), lambda i: (0, 0)),
        ],
        out_specs=pl.BlockSpec((BLK, D), lambda i: (i, 0)),
        out_shape=jax.ShapeDtypeStruct((NPAD, D), jnp.float32),
    )(parts1, g1, dinv, b1, w2)


def _kout_body(parts_ref, g2_ref, dinv_ref, b2_ref, out_ref):
    a = parts_ref[0] + parts_ref[1] + g2_ref[...]
    out_ref[...] = a * dinv_ref[...] + b2_ref[...]


def _kout(parts2, g2, dinv, b2):
    return pl.pallas_call(
        _kout_body,
        grid=(NPAD // BLK,),
        in_specs=[
            pl.BlockSpec((NC, BLK, D), lambda i: (0, i, 0)),
            pl.BlockSpec((BLK, D), lambda i: (i, 0)),
            pl.BlockSpec((BLK, 1), lambda i: (i, 0)),
            pl.BlockSpec((1, D), lambda i: (0, 0)),
        ],
        out_specs=pl.BlockSpec((BLK, D), lambda i: (i, 0)),
        out_shape=jax.ShapeDtypeStruct((NPAD, D), jnp.float32),
    )(parts2, g2, dinv, b2)


def kernel(x, edge_index, W1, b1, W2, b2):
    src = edge_index[0]
    dst = edge_index[1]
    pad = jnp.full((EPAD - E,), N, dtype=jnp.int32)
    srcr = jnp.concatenate([src.astype(jnp.int32), pad]).reshape(NW, NCHUNK, CH)
    dstr = jnp.concatenate([dst.astype(jnp.int32), pad]).reshape(NW, NCHUNK, CH)
    x_p = jnp.zeros((NPAD, D), jnp.float32).at[:N].set(x)
    ones_ch = jnp.ones((CH,), jnp.float32)
    zeros_rpt = jnp.zeros((RPT,), jnp.float32)
    zrows = jnp.zeros((CH, D), jnp.float32)

    deg_parts = _deg_kernel(dstr, ones_ch, zeros_rpt)
    g1, dinv = _k1(deg_parts, x_p, W1)
    parts1 = _agg_kernel(g1, srcr, dstr, zrows)
    g2 = _kmid(parts1, g1, dinv, b1.reshape(1, D), W2)
    parts2 = _agg_kernel(g2, srcr, dstr, zrows)
    out = _kout(parts2, g2, dinv, b2.reshape(1, D))
    return out[:N]
